# Initial kernel scaffold; baseline (speedup 1.0000x reference)
import jax
import jax.numpy as jnp
from jax.experimental import pallas as pl


def kernel(n_iters, vertex_attr, edgeij_pair, edge_attr, g):
    raise NotImplementedError("write your pallas kernel here")


# speedup vs baseline: 280.9092x; 280.9092x over previous
import functools

import jax
import jax.numpy as jnp
from jax import lax
from jax.experimental import pallas as pl
from jax.experimental.pallas import tpu as pltpu
from jax.experimental.pallas import tpu_sc as plsc

N_NODES = 100000
NPAD = 100352
SLICE = NPAD // 16
N_EDGES = 6400000
NW = 32
E_PER_W = N_EDGES // NW
K = 8000
CHUNKS = E_PER_W // K
GROUPS = K // 16

_mesh = plsc.VectorSubcoreMesh(core_axis_name="c", subcore_axis_name="s")


@functools.partial(
    pl.kernel,
    mesh=_mesh,
    out_type=jax.ShapeDtypeStruct((2, NPAD), jnp.float32),
    scratch_types=[
        pltpu.VMEM((NPAD,), jnp.float32),
        pltpu.VMEM((K,), jnp.int32),
        pltpu.VMEM((K,), jnp.int32),
        pltpu.VMEM((K,), jnp.float32),
        pltpu.VMEM_SHARED((NPAD,), jnp.float32),
    ],
    compiler_params=pltpu.CompilerParams(
        use_tc_tiling_on_sc=False, needs_layout_passes=False
    ),
)
def _spmv(x_hbm, col_hbm, row_hbm, a_hbm, out_hbm, xt, colv, rowv, av, csh):
    c = lax.axis_index("c")
    s = lax.axis_index("s")
    wid = c * 16 + s
    nsl = pl.ds(s * SLICE, SLICE)

    def _z(i, carry):
        av[pl.ds(i * 16, 16)] = jnp.zeros((16,), jnp.float32)
        return carry

    lax.fori_loop(0, SLICE // 16, _z, 0)
    pltpu.sync_copy(av.at[pl.ds(0, SLICE)], csh.at[nsl])
    pltpu.sync_copy(x_hbm, xt)
    plsc.subcore_barrier()

    ebase = wid * E_PER_W

    def _chunk(k, carry):
        off = ebase + k * K
        pltpu.sync_copy(col_hbm.at[pl.ds(off, K)], colv)
        pltpu.sync_copy(row_hbm.at[pl.ds(off, K)], rowv)
        pltpu.sync_copy(a_hbm.at[pl.ds(off, K)], av)

        def _mul(i, carry2):
            sl = pl.ds(i * 16, 16)
            xj = plsc.load_gather(xt, [colv[sl]])
            av[sl] = av[sl] * xj
            return carry2

        lax.fori_loop(0, GROUPS, _mul, 0)
        pltpu.sync_copy(av, csh.at[rowv], add=True)
        return carry

    lax.fori_loop(0, CHUNKS, _chunk, 0)
    plsc.subcore_barrier()
    pltpu.sync_copy(csh.at[nsl], out_hbm.at[c, nsl])


def _update_body(g_ref, x_ref, b_ref, ad_ref, p_ref, o_ref):
    w = g_ref[0, 0]
    cbar = p_ref[0] + p_ref[1]
    o_ref[...] = x_ref[...] + (w * (b_ref[...] - cbar)) / ad_ref[...]


def _update(x, b, adiag, part, g2):
    r = NPAD // 128
    out = pl.pallas_call(
        _update_body,
        out_shape=jax.ShapeDtypeStruct((r, 128), jnp.float32),
        in_specs=[
            pl.BlockSpec(memory_space=pltpu.SMEM),
            pl.BlockSpec(memory_space=pltpu.VMEM),
            pl.BlockSpec(memory_space=pltpu.VMEM),
            pl.BlockSpec(memory_space=pltpu.VMEM),
            pl.BlockSpec(memory_space=pltpu.VMEM),
        ],
        out_specs=pl.BlockSpec(memory_space=pltpu.VMEM),
    )(g2, x.reshape(r, 128), b.reshape(r, 128), adiag.reshape(r, 128),
      part.reshape(2, r, 128))
    return out.reshape(NPAD)


def kernel(n_iters, vertex_attr, edgeij_pair, edge_attr, g):
    row = edgeij_pair[0]
    col = edgeij_pair[1]
    a_e = edge_attr[:, 0]
    pad = NPAD - N_NODES
    adiag = jnp.pad(vertex_attr[:, 0], (0, pad), constant_values=1.0)
    b = jnp.pad(vertex_attr[:, 1], (0, pad))
    x0 = jnp.pad(vertex_attr[:, 2], (0, pad))
    g2 = g.reshape(1, 1)

    def body(_, x):
        part = _spmv(x, col, row, a_e)
        return _update(x, b, adiag, part, g2)

    x = lax.fori_loop(0, n_iters, body, x0)
    return x[:N_NODES].reshape(-1, 1)
